# baseline (speedup 1.0000x reference)
import functools

import jax
import jax.numpy as jnp
from jax import lax
from jax.experimental import pallas as pl
from jax.experimental.pallas import tpu as pltpu
from jax.experimental.pallas import tpu_sc as plsc

D_MODEL = 64
SCALE = 8.0
LANES = 16

NUM_CORES = 2
NUM_SUBCORES = 16
NUM_WORKERS = NUM_CORES * NUM_SUBCORES

IDX_W = 128
SUBS = 4
CHUNK = IDX_W * SUBS
NBUF = 2


def _emb_body(x_hbm, table_hbm, out_hbm, idx_v, rows_v, sems, *, n_chunks,
              per_w):
  wid = lax.axis_index("s") * NUM_CORES + lax.axis_index("c")
  base = wid * per_w

  pltpu.sync_copy(x_hbm.at[pl.ds(base, per_w)], idx_v)

  def fire(g, b):
    for j in range(SUBS):
      pltpu.async_copy(
          table_hbm.at[idx_v.at[pl.ds(g * CHUNK + j * IDX_W, IDX_W)]],
          rows_v.at[b].at[pl.ds(j * IDX_W, IDX_W)], sems[b])

  def finish(g, b):
    for j in range(SUBS):
      pltpu.make_async_copy(
          table_hbm.at[idx_v.at[pl.ds(g * CHUNK + j * IDX_W, IDX_W)]],
          rows_v.at[b].at[pl.ds(j * IDX_W, IDX_W)], sems[b]).wait()
    rv = rows_v.at[b]

    @plsc.parallel_loop(0, CHUNK, 1, unroll=8)
    def _(r):
      for l in range(D_MODEL // LANES):
        sl = pl.ds(l * LANES, LANES)
        rv[r, sl] = rv[r, sl] * SCALE

    pltpu.sync_copy(rv, out_hbm.at[pl.ds(base + g * CHUNK, CHUNK)])

  fire(0, 0)

  @pl.loop(0, n_chunks, step=NBUF)
  def _(g):
    fire(g + 1, 1)
    finish(g, 0)

    @pl.when(g + 2 < n_chunks)
    def _():
      fire(g + 2, 0)

    finish(g + 1, 1)


def kernel(x, table):
  b0, b1 = x.shape
  b = b0 * b1
  assert b % (NUM_WORKERS * CHUNK * NBUF) == 0
  per_w = b // NUM_WORKERS
  n_chunks = per_w // CHUNK

  mesh = plsc.VectorSubcoreMesh(core_axis_name="c", subcore_axis_name="s")
  emb = pl.kernel(
      functools.partial(_emb_body, n_chunks=n_chunks, per_w=per_w),
      out_type=jax.ShapeDtypeStruct((b, D_MODEL), jnp.float32),
      mesh=mesh,
      compiler_params=pltpu.CompilerParams(use_tc_tiling_on_sc=False),
      scratch_types=[
          pltpu.VMEM((per_w,), jnp.int32),
          pltpu.VMEM((NBUF, CHUNK, D_MODEL), jnp.float32),
          [pltpu.SemaphoreType.DMA for _ in range(NBUF)],
      ],
  )
  out = emb(x.reshape(b).astype(jnp.int32), table)
  return out.reshape(b0, b1, D_MODEL)


# speedup vs baseline: 1.0010x; 1.0010x over previous
import functools

import jax
import jax.numpy as jnp
from jax import lax
from jax.experimental import pallas as pl
from jax.experimental.pallas import tpu as pltpu
from jax.experimental.pallas import tpu_sc as plsc

D_MODEL = 64
SCALE = 8.0
LANES = 16

NUM_CORES = 2
NUM_SUBCORES = 16
NUM_WORKERS = NUM_CORES * NUM_SUBCORES

IDX_W = 128
SUBS = 2
CHUNK = IDX_W * SUBS
NBUF_G = 4
NBUF_W = 2


def _emb_body(x_hbm, table_hbm, out_hbm, idx_v, rows_v, wb_v, gsems, wsems,
              *, n_chunks, per_w):
  wid = lax.axis_index("s") * NUM_CORES + lax.axis_index("c")
  base = wid * per_w

  pltpu.sync_copy(x_hbm.at[pl.ds(base, per_w)], idx_v)

  def fire(g, b):
    for j in range(SUBS):
      pltpu.async_copy(
          table_hbm.at[idx_v.at[pl.ds(g * CHUNK + j * IDX_W, IDX_W)]],
          rows_v.at[b].at[pl.ds(j * IDX_W, IDX_W)], gsems[b])

  def wait_gathers(g, b):
    for j in range(SUBS):
      pltpu.make_async_copy(
          table_hbm.at[idx_v.at[pl.ds(g * CHUNK + j * IDX_W, IDX_W)]],
          rows_v.at[b].at[pl.ds(j * IDX_W, IDX_W)], gsems[b]).wait()

  def wb_descr(g, bw):
    return pltpu.make_async_copy(
        wb_v.at[bw], out_hbm.at[pl.ds(base + g * CHUNK, CHUNK)], wsems[bw])

  for b in range(NBUF_G):
    fire(b, b)

  @pl.loop(0, n_chunks, step=NBUF_G)
  def _(g):
    for k in range(NBUF_G):
      bg = k
      bw = k % NBUF_W
      gc = g + k
      wait_gathers(gc, bg)

      @pl.when(gc >= NBUF_W)
      def _():
        wb_descr(gc - NBUF_W, bw).wait()

      rv = rows_v.at[bg]
      wv = wb_v.at[bw]

      @plsc.parallel_loop(0, CHUNK, 1, unroll=8)
      def _(r):
        for l in range(D_MODEL // LANES):
          sl = pl.ds(l * LANES, LANES)
          wv[r, sl] = rv[r, sl] * SCALE

      pltpu.async_copy(wb_v.at[bw],
                       out_hbm.at[pl.ds(base + gc * CHUNK, CHUNK)], wsems[bw])

      @pl.when(gc + NBUF_G < n_chunks)
      def _():
        fire(gc + NBUF_G, bg)

  for gl in range(n_chunks - NBUF_W, n_chunks):
    wb_descr(gl, gl % NBUF_W).wait()


def kernel(x, table):
  b0, b1 = x.shape
  b = b0 * b1
  assert b % (NUM_WORKERS * CHUNK * NBUF_G) == 0
  per_w = b // NUM_WORKERS
  n_chunks = per_w // CHUNK

  mesh = plsc.VectorSubcoreMesh(core_axis_name="c", subcore_axis_name="s")
  emb = pl.kernel(
      functools.partial(_emb_body, n_chunks=n_chunks, per_w=per_w),
      out_type=jax.ShapeDtypeStruct((b, D_MODEL), jnp.float32),
      mesh=mesh,
      compiler_params=pltpu.CompilerParams(use_tc_tiling_on_sc=False),
      scratch_types=[
          pltpu.VMEM((per_w,), jnp.int32),
          pltpu.VMEM((NBUF_G, CHUNK, D_MODEL), jnp.float32),
          pltpu.VMEM((NBUF_W, CHUNK, D_MODEL), jnp.float32),
          [pltpu.SemaphoreType.DMA for _ in range(NBUF_G)],
          [pltpu.SemaphoreType.DMA for _ in range(NBUF_W)],
      ],
  )
  xf = x.reshape(b)
  if xf.dtype != jnp.int32:
    xf = xf.astype(jnp.int32)
  return emb(xf, table).reshape(b0, b1, D_MODEL)
